# baseline (speedup 1.0000x reference)
import functools

import jax
import jax.numpy as jnp
import numpy as np
from jax import lax
from jax.experimental import pallas as pl
from jax.experimental.pallas import tpu as pltpu

_NB = 8
_TQ = 4


def _seg_avg_const(f, bw):
    blk = np.zeros((bw, bw), np.float32)
    blk[:f, :f] = 1.0 / f
    return np.kron(np.eye(_NB, dtype=np.float32), blk)


def _stencil_coeffs(t_len, dtype):
    t = lax.broadcasted_iota(jnp.int32, (t_len, 1), 0)
    inv_s2 = 0.7071067811865475
    inv_s3 = 0.5773502691896258

    def dis(s):
        edge = (s == 0) | (s == t_len - 1)
        return jnp.where(edge, inv_s2, inv_s3).astype(dtype)

    d0 = dis(t)
    c_self = d0 * d0
    c_prev = jnp.where(t >= 1, dis(t - 1), 0.0).astype(dtype) * d0
    c_next = jnp.where(t <= t_len - 2, dis(t + 1), 0.0).astype(dtype) * d0
    return c_self, c_prev, c_next


def _layer(h, m_ref, s_ref, ba_ref, g_ref, be_ref, c_self, c_prev, c_next):
    u = jnp.dot(h.astype(jnp.bfloat16), m_ref[...],
                preferred_element_type=jnp.float32)
    hc = (c_self * u + c_prev * jnp.roll(u, 1, axis=0)
          + c_next * jnp.roll(u, -1, axis=0) + ba_ref[...])
    v = jnp.dot((hc * hc).astype(jnp.bfloat16), s_ref[...],
                preferred_element_type=jnp.float32)
    return jnp.maximum(hc * lax.rsqrt(v + 1e-5) * g_ref[...] + be_ref[...],
                       0.0)


def _encoder_kernel(xp_ref,
                    m1_ref, s1_ref, ba1_ref, g1_ref, be1_ref,
                    m2_ref, s2_ref, ba2_ref, g2_ref, be2_ref,
                    m3_ref, s3_ref, ba3_ref, g3_ref, be3_ref,
                    wo_ref, bo_ref, out_ref, h3_ref, *, t_len, latent):
    j = pl.program_id(1)

    @pl.when(j == 0)
    def _compute_layers():
        c = _stencil_coeffs(t_len, jnp.float32)
        h = xp_ref[0]
        h = _layer(h, m1_ref, s1_ref, ba1_ref, g1_ref, be1_ref, *c)
        h = _layer(h, m2_ref, s2_ref, ba2_ref, g2_ref, be2_ref, *c)
        h = _layer(h, m3_ref, s3_ref, ba3_ref, g3_ref, be3_ref, *c)
        h3_ref[...] = h.astype(jnp.bfloat16)

    rows = t_len // _TQ
    hb = h3_ref[pl.ds(j * rows, rows), :]
    oa = jnp.dot(hb, wo_ref[...], preferred_element_type=jnp.float32)
    for s in range(_NB):
        out_ref[s] = oa[:, s * latent:(s + 1) * latent] + bo_ref[...]


def _blk_weight(wt, bw_in, bw_out):
    wt = wt - jnp.mean(wt, axis=1, keepdims=True)
    wt = jnp.pad(wt, ((0, bw_in - wt.shape[0]), (0, bw_out - wt.shape[1])))
    return jnp.kron(jnp.eye(_NB, dtype=wt.dtype), wt).astype(jnp.bfloat16)


def _blk_vec(v, bw, center=False):
    if center:
        v = v - jnp.mean(v)
    return jnp.tile(jnp.pad(v, (0, bw - v.shape[0])), _NB)[None, :]


@functools.partial(jax.jit, static_argnames=("interpret",))
def _run(x, W1, b1, g1, be1, W2, b2, g2, be2, W3, b3, g3, be3, Wo, bo,
         interpret=False):
    b_, t_, d_in = x.shape
    latent = Wo.shape[0]
    nblk = b_ // _NB
    rows = t_ // _TQ
    xp = x.reshape(nblk, _NB, t_, d_in).transpose(0, 2, 1, 3)
    xp = xp.reshape(nblk, t_, _NB * d_in)

    m1 = _blk_weight(W1.T, d_in, 16)
    m2 = _blk_weight(W2.T, 16, 16)
    m3 = _blk_weight(W3.T, 16, 32)
    s1 = jnp.asarray(_seg_avg_const(12, 16), dtype=jnp.bfloat16)
    s2 = s1
    s3 = jnp.asarray(_seg_avg_const(24, 32), dtype=jnp.bfloat16)
    ba1 = _blk_vec(b1, 16, center=True)
    ba2 = _blk_vec(b2, 16, center=True)
    ba3 = _blk_vec(b3, 32, center=True)
    g1b, be1b = _blk_vec(g1, 16), _blk_vec(be1, 16)
    g2b, be2b = _blk_vec(g2, 16), _blk_vec(be2, 16)
    g3b, be3b = _blk_vec(g3, 32), _blk_vec(be3, 32)
    wo_big = jnp.kron(jnp.eye(_NB, dtype=Wo.dtype),
                      jnp.pad(Wo.T, ((0, 8), (0, 0)))).astype(jnp.bfloat16)
    bo2 = bo[None, :]

    def xmap(i, j):
        return (i, 0, 0)

    def omap(i, j):
        return (i, j, 0)

    def wmap(i, j):
        return (0, 0)

    params = [m1, s1, ba1, g1b, be1b,
              m2, s2, ba2, g2b, be2b,
              m3, s3, ba3, g3b, be3b,
              wo_big, bo2]
    param_specs = [pl.BlockSpec(p.shape, wmap) for p in params]

    return pl.pallas_call(
        functools.partial(_encoder_kernel, t_len=t_, latent=latent),
        grid=(nblk, _TQ),
        in_specs=[pl.BlockSpec((1, t_, _NB * d_in), xmap)] + param_specs,
        out_specs=pl.BlockSpec((_NB, rows, latent), omap),
        out_shape=jax.ShapeDtypeStruct((b_, t_, latent), jnp.float32),
        scratch_shapes=[pltpu.VMEM((t_, _NB * 32), jnp.bfloat16)],
        interpret=interpret,
    )(xp, *params)


def kernel(x, W1, b1, g1, be1, W2, b2, g2, be2, W3, b3, g3, be3, Wo, bo):
    return _run(x, W1, b1, g1, be1, W2, b2, g2, be2, W3, b3, g3, be3, Wo, bo)


# speedup vs baseline: 1.5207x; 1.5207x over previous
import functools

import jax
import jax.numpy as jnp
import numpy as np
from jax import lax
from jax.experimental import pallas as pl
from jax.experimental.pallas import tpu as pltpu

_NB = 8
_TQ = 4


def _seg_avg_const(f, bw):
    blk = np.zeros((bw, bw), np.float32)
    blk[:f, :f] = 1.0 / f
    return np.kron(np.eye(_NB, dtype=np.float32), blk)


def _stencil_coeffs(t_len, dtype):
    t = lax.broadcasted_iota(jnp.int32, (t_len, 1), 0)
    inv_s2 = 0.7071067811865475
    inv_s3 = 0.5773502691896258

    def dis(s):
        edge = (s == 0) | (s == t_len - 1)
        return jnp.where(edge, inv_s2, inv_s3).astype(dtype)

    d0 = dis(t)
    c_self = d0 * d0
    c_prev = jnp.where(t >= 1, dis(t - 1), 0.0).astype(dtype) * d0
    c_next = jnp.where(t <= t_len - 2, dis(t + 1), 0.0).astype(dtype) * d0
    return c_self, c_prev, c_next


def _layer(h, m_ref, s_ref, ba_ref, g_ref, be_ref, c_self, c_prev, c_next):
    u = jnp.dot(h.astype(jnp.bfloat16), m_ref[...],
                preferred_element_type=jnp.float32)
    hc = (c_self * u + c_prev * jnp.roll(u, 1, axis=0)
          + c_next * jnp.roll(u, -1, axis=0) + ba_ref[...])
    v = jnp.dot((hc * hc).astype(jnp.bfloat16), s_ref[...],
                preferred_element_type=jnp.float32)
    return jnp.maximum(hc * lax.rsqrt(v + 1e-5) * g_ref[...] + be_ref[...],
                       0.0)


def _encoder_kernel(xp_ref,
                    m1_ref, s1_ref, ba1_ref, g1_ref, be1_ref,
                    m2_ref, s2_ref, ba2_ref, g2_ref, be2_ref,
                    m3_ref, s3_ref, ba3_ref, g3_ref, be3_ref,
                    wo_ref, bo_ref, out_ref, *, t_len, latent):
    c = _stencil_coeffs(t_len, jnp.float32)
    h = xp_ref[0]
    h = _layer(h, m1_ref, s1_ref, ba1_ref, g1_ref, be1_ref, *c)
    h = _layer(h, m2_ref, s2_ref, ba2_ref, g2_ref, be2_ref, *c)
    h = _layer(h, m3_ref, s3_ref, ba3_ref, g3_ref, be3_ref, *c)
    oa = jnp.dot(h.astype(jnp.bfloat16), wo_ref[...],
                 preferred_element_type=jnp.float32)
    for s in range(_NB):
        out_ref[s] = oa[:, s * latent:(s + 1) * latent] + bo_ref[...]


def _blk_weight(wt, bw_in, bw_out):
    wt = wt - jnp.mean(wt, axis=1, keepdims=True)
    wt = jnp.pad(wt, ((0, bw_in - wt.shape[0]), (0, bw_out - wt.shape[1])))
    return jnp.kron(jnp.eye(_NB, dtype=wt.dtype), wt).astype(jnp.bfloat16)


def _blk_vec(v, bw, center=False):
    if center:
        v = v - jnp.mean(v)
    return jnp.tile(jnp.pad(v, (0, bw - v.shape[0])), _NB)[None, :]


@functools.partial(jax.jit, static_argnames=("interpret",))
def _run(x, W1, b1, g1, be1, W2, b2, g2, be2, W3, b3, g3, be3, Wo, bo,
         interpret=False):
    b_, t_, d_in = x.shape
    latent = Wo.shape[0]
    nblk = b_ // _NB
    rows = t_ // _TQ
    xp = x.reshape(nblk, _NB, t_, d_in).transpose(0, 2, 1, 3)
    xp = xp.reshape(nblk, t_, _NB * d_in)

    m1 = _blk_weight(W1.T, d_in, 16)
    m2 = _blk_weight(W2.T, 16, 16)
    m3 = _blk_weight(W3.T, 16, 32)
    s1 = jnp.asarray(_seg_avg_const(12, 16), dtype=jnp.bfloat16)
    s2 = s1
    s3 = jnp.asarray(_seg_avg_const(24, 32), dtype=jnp.bfloat16)
    ba1 = _blk_vec(b1, 16, center=True)
    ba2 = _blk_vec(b2, 16, center=True)
    ba3 = _blk_vec(b3, 32, center=True)
    g1b, be1b = _blk_vec(g1, 16), _blk_vec(be1, 16)
    g2b, be2b = _blk_vec(g2, 16), _blk_vec(be2, 16)
    g3b, be3b = _blk_vec(g3, 32), _blk_vec(be3, 32)
    wo_big = jnp.kron(jnp.eye(_NB, dtype=Wo.dtype),
                      jnp.pad(Wo.T, ((0, 8), (0, 0)))).astype(jnp.bfloat16)
    bo2 = bo[None, :]

    def xmap(i):
        return (i, 0, 0)

    def wmap(i):
        return (0, 0)

    params = [m1, s1, ba1, g1b, be1b,
              m2, s2, ba2, g2b, be2b,
              m3, s3, ba3, g3b, be3b,
              wo_big, bo2]
    param_specs = [pl.BlockSpec(p.shape, wmap) for p in params]

    return pl.pallas_call(
        functools.partial(_encoder_kernel, t_len=t_, latent=latent),
        grid=(nblk,),
        in_specs=[pl.BlockSpec((1, t_, _NB * d_in), xmap)] + param_specs,
        out_specs=pl.BlockSpec((_NB, t_, latent), xmap),
        out_shape=jax.ShapeDtypeStruct((b_, t_, latent), jnp.float32),
        interpret=interpret,
    )(xp, *params)


def kernel(x, W1, b1, g1, be1, W2, b2, g2, be2, W3, b3, g3, be3, Wo, bo):
    return _run(x, W1, b1, g1, be1, W2, b2, g2, be2, W3, b3, g3, be3, Wo, bo)
